# baseline (speedup 1.0000x reference)
import jax
import jax.numpy as jnp
from jax.experimental import pallas as pl

N = 10000
E = 320000
K = 10
ALPHA = 0.1


def _logsoftmax_body(h_ref, o_ref):
    h = h_ref[...]
    m = jnp.max(h, axis=-1, keepdims=True)
    e = jnp.exp(h - m)
    o_ref[...] = h - m - jnp.log(jnp.sum(e, axis=-1, keepdims=True))


def kernel(x, edge_index, edge_weight, W1, b1, Wc, bc, W2, b2):
    row = edge_index[0]
    col = edge_index[1]
    loop = jnp.arange(N, dtype=row.dtype)
    rowf = jnp.concatenate([row, loop])
    colf = jnp.concatenate([col, loop])
    ew = jnp.concatenate([edge_weight, jnp.ones((N,), dtype=edge_weight.dtype)])
    deg = jnp.zeros((N,), dtype=ew.dtype).at[colf].add(ew)
    dinv = jax.lax.rsqrt(deg)
    norm = dinv[rowf] * ew * dinv[colf]

    def prop(h):
        msg = norm[:, None] * h[rowf]
        return jnp.zeros((N, h.shape[1]), dtype=h.dtype).at[colf].add(msg)

    h = jax.nn.relu(x @ W1 + b1)
    h = jax.nn.relu(prop(h @ Wc) + bc)
    h = h @ W2 + b2
    x0 = h
    for _ in range(K):
        h = prop(h)
        h = h * (1.0 - ALPHA) + ALPHA * x0

    return pl.pallas_call(
        _logsoftmax_body,
        out_shape=jax.ShapeDtypeStruct(h.shape, h.dtype),
    )(h)


# speedup vs baseline: 9.9541x; 9.9541x over previous
import functools

import jax
import jax.numpy as jnp
from jax import lax
from jax.experimental import pallas as pl
from jax.experimental.pallas import tpu as pltpu
from jax.experimental.pallas import tpu_sc as plsc

N = 10000
E = 320000
K = 10
ALPHA = 0.1

NC = 2
NS = 16
NW = NC * NS
B = 128
NCH = 80
EPW = NCH * B
EP = NW * EPW
RPS = N // NS
ZR = 125
NP = 10240
CW = NP // NS

_MESH = plsc.VectorSubcoreMesh(
    core_axis_name="c", subcore_axis_name="s", num_cores=NC, num_subcores=NS
)
_SC_PARAMS = pltpu.CompilerParams(use_tc_tiling_on_sc=False, needs_layout_passes=False)


def _worker(c, s):
    return s * NC + c


def _run_edges(F, BF, nch, pk_v, gsrc, shared, nrmbuf, idxcS, rows, rowsS, sem):
    def chunk(j, carry):
        pltpu.async_copy(gsrc.at[pk_v.at[j, 0]], rows, sem).wait()
        for i in range(BF // 16):
            sl = pl.ds(i * 16, 16)
            nrmbuf[sl] = plsc.bitcast(pk_v[j, 2, sl], jnp.float32)
            idxcS[sl] = pk_v[j, 1, sl]
        for i in range(BF):
            sp = plsc.load_gather(nrmbuf, [jnp.full((16,), i, jnp.int32)])
            for qq in range(F // 16):
                sl = pl.ds(qq * 16, 16)
                rowsS[i, sl] = rows[i, sl] * sp
        pltpu.sync_copy(rowsS, shared.at[idxcS], add=True)
        return carry

    lax.fori_loop(0, nch, chunk, 0)


def _zero_shared(F, s, zbuf, shared):
    def zb(i, carry):
        for q in range(F // 16):
            zbuf[i, pl.ds(q * 16, 16)] = jnp.zeros((16,), jnp.float32)
        return carry

    lax.fori_loop(0, ZR, zb, 0)
    for r in range(RPS // ZR):
        pltpu.sync_copy(zbuf, shared.at[pl.ds(s * RPS + r * ZR, ZR)])


def _prop16_body(h_hbm, pk_hbm, out_hbm,
                 pk_v, nrmbuf, idxcS, rows, rowsS, zbuf, shared, shared_h, sem):
    c = lax.axis_index("c")
    s = lax.axis_index("s")
    w = _worker(c, s)
    pltpu.sync_copy(pk_hbm.at[w], pk_v)
    pltpu.sync_copy(h_hbm.at[pl.ds(s * RPS, RPS)],
                    shared_h.at[pl.ds(s * RPS, RPS)])
    _zero_shared(16, s, zbuf, shared)
    plsc.subcore_barrier()
    _run_edges(16, B, NCH, pk_v, shared_h, shared, nrmbuf, idxcS, rows,
               rowsS, sem)
    plsc.subcore_barrier()
    pltpu.sync_copy(shared.at[pl.ds(s * RPS, RPS)],
                    out_hbm.at[c, pl.ds(s * RPS, RPS)])


_prop16 = pl.kernel(
    _prop16_body,
    out_type=jax.ShapeDtypeStruct((NC, N, 16), jnp.float32),
    mesh=_MESH,
    scratch_types=[
        pltpu.VMEM((NCH, 3, B), jnp.int32),
        pltpu.VMEM((B,), jnp.float32),
        pltpu.VMEM((B,), jnp.int32),
        pltpu.VMEM((B, 16), jnp.float32),
        pltpu.VMEM((B, 16), jnp.float32),
        pltpu.VMEM((ZR, 16), jnp.float32),
        pltpu.VMEM_SHARED((N, 16), jnp.float32),
        pltpu.VMEM_SHARED((N, 16), jnp.float32),
        pltpu.SemaphoreType.DMA,
    ],
    compiler_params=_SC_PARAMS,
)

BF2 = 64
FW = 64
NCH2 = EPW // BF2


def _prop256_body(gcat_hbm, pk4_hbm, out_hbm,
                  pk_v, nrmbuf, idxcS, rows, rowsS, zbuf, shared, sem):
    c = lax.axis_index("c")
    s = lax.axis_index("s")
    w = _worker(c, s)

    def one_pass(p, carry):
        pltpu.sync_copy(pk4_hbm.at[p, w], pk_v)
        _zero_shared(FW, s, zbuf, shared)
        plsc.subcore_barrier()
        _run_edges(FW, BF2, NCH2, pk_v, gcat_hbm, shared, nrmbuf, idxcS,
                   rows, rowsS, sem)
        plsc.subcore_barrier()
        pltpu.sync_copy(shared.at[pl.ds(s * RPS, RPS)],
                        out_hbm.at[c, p, pl.ds(s * RPS, RPS)])
        plsc.subcore_barrier()
        return carry

    lax.fori_loop(0, 4, one_pass, 0)


_prop256 = pl.kernel(
    _prop256_body,
    out_type=jax.ShapeDtypeStruct((NC, 4, N, FW), jnp.float32),
    mesh=_MESH,
    scratch_types=[
        pltpu.VMEM((NCH2, 3, BF2), jnp.int32),
        pltpu.VMEM((BF2,), jnp.float32),
        pltpu.VMEM((BF2,), jnp.int32),
        pltpu.VMEM((BF2, FW), jnp.float32),
        pltpu.VMEM((BF2, FW), jnp.float32),
        pltpu.VMEM((ZR, FW), jnp.float32),
        pltpu.VMEM_SHARED((N, FW), jnp.float32),
        pltpu.SemaphoreType.DMA,
    ],
    compiler_params=_SC_PARAMS,
)


def _deg_body(col_hbm, ew_hbm, out_hbm, idx_v, ew_v, deg_v, tmp_v, acc_v, shared):
    c = lax.axis_index("c")
    s = lax.axis_index("s")
    w = _worker(c, s)

    def z(i, carry):
        deg_v[pl.ds(i * 16, 16)] = jnp.zeros((16,), jnp.float32)
        return carry

    lax.fori_loop(0, NP // 16, z, 0)

    def chunk(j, carry):
        base = w * EPW + j * B
        pltpu.sync_copy(col_hbm.at[pl.ds(base, B)], idx_v)
        pltpu.sync_copy(ew_hbm.at[pl.ds(base, B)], ew_v)
        for i in range(B // 16):
            sl = pl.ds(i * 16, 16)
            plsc.addupdate_scatter(deg_v, [idx_v[sl]], ew_v[sl])
        return carry

    lax.fori_loop(0, NCH, chunk, 0)

    pltpu.sync_copy(deg_v, shared.at[s])
    plsc.subcore_barrier()

    def z2(i, carry):
        acc_v[pl.ds(i * 16, 16)] = jnp.zeros((16,), jnp.float32)
        return carry

    lax.fori_loop(0, CW // 16, z2, 0)
    for t in range(NS):
        pltpu.sync_copy(shared.at[t, pl.ds(s * CW, CW)], tmp_v)
        for i in range(CW // 16):
            sl = pl.ds(i * 16, 16)
            acc_v[sl] = acc_v[sl] + tmp_v[sl]
    pltpu.sync_copy(acc_v, out_hbm.at[c, pl.ds(s * CW, CW)])


_sc_deg = pl.kernel(
    _deg_body,
    out_type=jax.ShapeDtypeStruct((NC, NP), jnp.float32),
    mesh=_MESH,
    scratch_types=[
        pltpu.VMEM((B,), jnp.int32),
        pltpu.VMEM((B,), jnp.float32),
        pltpu.VMEM((NP,), jnp.float32),
        pltpu.VMEM((CW,), jnp.float32),
        pltpu.VMEM((CW,), jnp.float32),
        pltpu.VMEM_SHARED((NS, NP), jnp.float32),
    ],
    compiler_params=_SC_PARAMS,
)


def _norm_body(row_hbm, col_hbm, ew_hbm, dinv_hbm, out_hbm,
               idxr_v, idxc_v, ew_v, nrm_v, dinv_v):
    c = lax.axis_index("c")
    s = lax.axis_index("s")
    w = _worker(c, s)
    pltpu.sync_copy(dinv_hbm, dinv_v)

    def chunk(j, carry):
        base = w * EPW + j * B
        pltpu.sync_copy(row_hbm.at[pl.ds(base, B)], idxr_v)
        pltpu.sync_copy(col_hbm.at[pl.ds(base, B)], idxc_v)
        pltpu.sync_copy(ew_hbm.at[pl.ds(base, B)], ew_v)
        for i in range(B // 16):
            sl = pl.ds(i * 16, 16)
            vr = plsc.load_gather(dinv_v, [idxr_v[sl]])
            vc = plsc.load_gather(dinv_v, [idxc_v[sl]])
            nrm_v[sl] = vr * ew_v[sl] * vc
        pltpu.sync_copy(nrm_v, out_hbm.at[pl.ds(base, B)])
        return carry

    lax.fori_loop(0, NCH, chunk, 0)


_sc_norm = pl.kernel(
    _norm_body,
    out_type=jax.ShapeDtypeStruct((EP,), jnp.float32),
    mesh=_MESH,
    scratch_types=[
        pltpu.VMEM((B,), jnp.int32),
        pltpu.VMEM((B,), jnp.int32),
        pltpu.VMEM((B,), jnp.float32),
        pltpu.VMEM((B,), jnp.float32),
        pltpu.VMEM((NP,), jnp.float32),
    ],
    compiler_params=_SC_PARAMS,
)


def _prep_body(deg2_ref, dinv_ref, sn_ref):
    d = deg2_ref[0] + deg2_ref[1] + 1.0
    dinv_ref[...] = lax.rsqrt(d)
    sn_ref[...] = 1.0 / d


_tc_prep = pl.pallas_call(
    _prep_body,
    out_shape=[
        jax.ShapeDtypeStruct((NP // 128, 128), jnp.float32),
        jax.ShapeDtypeStruct((NP // 128, 128), jnp.float32),
    ],
)

BR = 400


def _head_body(x_ref, W1_ref, b1_ref, Wc_ref, g0_ref, g1_ref, g2_ref, g3_ref):
    h1 = jnp.dot(x_ref[...], W1_ref[...], preferred_element_type=jnp.float32)
    h1 = jnp.maximum(h1 + b1_ref[...], 0.0)
    g = jnp.dot(h1, Wc_ref[...], preferred_element_type=jnp.float32)
    for k, r in enumerate((g0_ref, g1_ref, g2_ref, g3_ref)):
        r[...] = g[:, k * 64:(k + 1) * 64]


_tc_head = pl.pallas_call(
    _head_body,
    grid=(N // BR,),
    in_specs=[
        pl.BlockSpec((BR, 128), lambda i: (i, 0)),
        pl.BlockSpec((128, 256), lambda i: (0, 0)),
        pl.BlockSpec((1, 256), lambda i: (0, 0)),
        pl.BlockSpec((256, 256), lambda i: (0, 0)),
    ],
    out_specs=[pl.BlockSpec((BR, 64), lambda i: (i, 0)) for _ in range(4)],
    out_shape=[jax.ShapeDtypeStruct((N, 64), jnp.float32) for _ in range(4)],
)


def _mid_body(p0_ref, p1_ref, p2_ref, p3_ref, g0_ref, g1_ref, g2_ref, g3_ref,
              sn_ref, bc_ref, W2_ref, b2_ref, o_ref):
    sn = sn_ref[...]
    parts = [p_ref[0, 0] + p_ref[1, 0] + sn * g_ref[...]
             for p_ref, g_ref in zip((p0_ref, p1_ref, p2_ref, p3_ref),
                                     (g0_ref, g1_ref, g2_ref, g3_ref))]
    h2 = jnp.maximum(jnp.concatenate(parts, axis=1) + bc_ref[...], 0.0)
    o_ref[...] = jnp.dot(h2, W2_ref[...], preferred_element_type=jnp.float32) + b2_ref[...]


_tc_mid = pl.pallas_call(
    _mid_body,
    grid=(N // BR,),
    in_specs=[
        pl.BlockSpec((2, 1, BR, 64), lambda i, k=k: (0, k, i, 0))
        for k in range(4)
    ] + [
        pl.BlockSpec((BR, 64), lambda i: (i, 0)) for _ in range(4)
    ] + [
        pl.BlockSpec((BR, 1), lambda i: (i, 0)),
        pl.BlockSpec((1, 256), lambda i: (0, 0)),
        pl.BlockSpec((256, 16), lambda i: (0, 0)),
        pl.BlockSpec((1, 16), lambda i: (0, 0)),
    ],
    out_specs=pl.BlockSpec((BR, 16), lambda i: (i, 0)),
    out_shape=jax.ShapeDtypeStruct((N, 16), jnp.float32),
)


def _axpy_body(p_ref, sn_ref, h_ref, x0_ref, o_ref):
    o_ref[...] = (1.0 - ALPHA) * (p_ref[0] + p_ref[1] + sn_ref[...] * h_ref[...]) \
        + ALPHA * x0_ref[...]


_tc_axpy = pl.pallas_call(
    _axpy_body,
    out_shape=jax.ShapeDtypeStruct((N * 16 // 128, 128), jnp.float32),
)


def _final_body(p_ref, sn_ref, h_ref, x0_ref, o_ref):
    t = (1.0 - ALPHA) * (p_ref[0] + p_ref[1] + sn_ref[...] * h_ref[...]) \
        + ALPHA * x0_ref[...]
    m = jnp.max(t, axis=-1, keepdims=True)
    e = jnp.exp(t - m)
    o_ref[...] = t - m - jnp.log(jnp.sum(e, axis=-1, keepdims=True))


_tc_final = pl.pallas_call(
    _final_body,
    out_shape=jax.ShapeDtypeStruct((N, 16), jnp.float32),
)


def kernel(x, edge_index, edge_weight, W1, b1, Wc, bc, W2, b2):
    row = edge_index[0]
    col = edge_index[1]
    pad = EP - E
    rowp = jnp.concatenate([row, jnp.zeros((pad,), jnp.int32)])
    colp = jnp.concatenate([col, jnp.zeros((pad,), jnp.int32)])
    ewp = jnp.concatenate([edge_weight, jnp.zeros((pad,), jnp.float32)])

    deg2 = _sc_deg(colp, ewp)
    dinv80, sn80 = _tc_prep(deg2.reshape(NC, NP // 128, 128))
    dinv = dinv80.reshape(NP)
    snf = sn80.reshape(NP)[:N]
    sn_col = snf[:, None]
    sn16 = jnp.broadcast_to(sn_col, (N, 16))
    sn16r = sn16.reshape(N * 16 // 128, 128)

    normp = _sc_norm(rowp, colp, ewp, dinv)
    nbits = lax.bitcast_convert_type(normp, jnp.int32)

    pk16 = jnp.stack([rowp.reshape(NW, NCH, B), colp.reshape(NW, NCH, B),
                      nbits.reshape(NW, NCH, B)], axis=2)
    rowp4 = rowp[None, :] + (jnp.arange(4, dtype=jnp.int32) * N)[:, None]
    c4 = jnp.broadcast_to(colp.reshape(1, NW, NCH2, BF2), (4, NW, NCH2, BF2))
    n4 = jnp.broadcast_to(nbits.reshape(1, NW, NCH2, BF2), (4, NW, NCH2, BF2))
    pk256 = jnp.stack([rowp4.reshape(4, NW, NCH2, BF2), c4, n4], axis=3)

    gq = _tc_head(x, W1, b1.reshape(1, -1), Wc)
    gcat = jnp.concatenate(gq, axis=0)
    p256 = _prop256(gcat, pk256)
    x0 = _tc_mid(p256, p256, p256, p256, gq[0], gq[1], gq[2], gq[3],
                 sn_col, bc.reshape(1, -1), W2, b2.reshape(1, -1))

    h = x0
    x0r = x0.reshape(N * 16 // 128, 128)
    for _ in range(K - 1):
        p = _prop16(h, pk16)
        hr = _tc_axpy(p.reshape(2, N * 16 // 128, 128), sn16r,
                      h.reshape(N * 16 // 128, 128), x0r)
        h = hr.reshape(N, 16)
    p = _prop16(h, pk16)
    return _tc_final(p, sn16, h, x0)


# speedup vs baseline: 12.4180x; 1.2475x over previous
import functools

import jax
import jax.numpy as jnp
from jax import lax
from jax.experimental import pallas as pl
from jax.experimental.pallas import tpu as pltpu
from jax.experimental.pallas import tpu_sc as plsc

N = 10000
E = 320000
K = 10
ALPHA = 0.1

NC = 2
NS = 16
NW = NC * NS
B = 128
NCH = 80
EPW = NCH * B
EP = NW * EPW
RPS = N // NS
ZR = 125
NP = 10240
CW = NP // NS

_MESH = plsc.VectorSubcoreMesh(
    core_axis_name="c", subcore_axis_name="s", num_cores=NC, num_subcores=NS
)
_SC_PARAMS = pltpu.CompilerParams(use_tc_tiling_on_sc=False, needs_layout_passes=False)


def _worker(c, s):
    return s * NC + c


def _run_edges(F, BF, nch, pk_v, gsrc, shared, nrmbuf, idxcS, rows, rowsS,
               semG):
    def phase(j, p):
        q = 1 - p
        pltpu.make_async_copy(
            gsrc.at[pk_v.at[lax.rem(j, nch), 0]], rows[p], semG[p]).wait()
        pltpu.async_copy(
            gsrc.at[pk_v.at[lax.rem(j + 1, nch), 0]], rows[q], semG[q])
        for i in range(BF // 16):
            sl = pl.ds(i * 16, 16)
            nrmbuf[sl] = plsc.bitcast(pk_v[j, 2, sl], jnp.float32)
            idxcS[sl] = pk_v[j, 1, sl]
        for i in range(BF):
            sp = plsc.load_gather(nrmbuf, [jnp.full((16,), i, jnp.int32)])
            for qq in range(F // 16):
                sl = pl.ds(qq * 16, 16)
                rowsS[i, sl] = rows[p][i, sl] * sp
        pltpu.sync_copy(rowsS, shared.at[idxcS], add=True)

    pltpu.async_copy(gsrc.at[pk_v.at[0, 0]], rows[0], semG[0])

    def steady(jj, carry):
        j = 2 * jj
        phase(j, 0)
        phase(j + 1, 1)
        return carry

    lax.fori_loop(0, nch // 2, steady, 0)
    pltpu.make_async_copy(gsrc.at[pk_v.at[0, 0]], rows[0], semG[0]).wait()


def _zero_shared(F, s, zbuf, shared):
    def zb(i, carry):
        for q in range(F // 16):
            zbuf[i, pl.ds(q * 16, 16)] = jnp.zeros((16,), jnp.float32)
        return carry

    lax.fori_loop(0, ZR, zb, 0)
    for r in range(RPS // ZR):
        pltpu.sync_copy(zbuf, shared.at[pl.ds(s * RPS + r * ZR, ZR)])


def _prop16_body(h_hbm, pk_hbm, out_hbm,
                 pk_v, nrmbuf, idxcS, rows0, rows1, rowsS, zbuf, shared,
                 shared_h, sem0, sem1):
    c = lax.axis_index("c")
    s = lax.axis_index("s")
    w = _worker(c, s)
    pltpu.sync_copy(pk_hbm.at[w], pk_v)
    pltpu.sync_copy(h_hbm.at[pl.ds(s * RPS, RPS)],
                    shared_h.at[pl.ds(s * RPS, RPS)])
    _zero_shared(16, s, zbuf, shared)
    plsc.subcore_barrier()
    _run_edges(16, B, NCH, pk_v, shared_h, shared, nrmbuf, idxcS,
               (rows0, rows1), rowsS, (sem0, sem1))
    plsc.subcore_barrier()
    pltpu.sync_copy(shared.at[pl.ds(s * RPS, RPS)],
                    out_hbm.at[c, pl.ds(s * RPS, RPS)])


_prop16 = pl.kernel(
    _prop16_body,
    out_type=jax.ShapeDtypeStruct((NC, N, 16), jnp.float32),
    mesh=_MESH,
    scratch_types=[
        pltpu.VMEM((NCH, 3, B), jnp.int32),
        pltpu.VMEM((B,), jnp.float32),
        pltpu.VMEM((B,), jnp.int32),
        pltpu.VMEM((B, 16), jnp.float32),
        pltpu.VMEM((B, 16), jnp.float32),
        pltpu.VMEM((B, 16), jnp.float32),
        pltpu.VMEM((ZR, 16), jnp.float32),
        pltpu.VMEM_SHARED((N, 16), jnp.float32),
        pltpu.VMEM_SHARED((N, 16), jnp.float32),
        pltpu.SemaphoreType.DMA, pltpu.SemaphoreType.DMA,
    ],
    compiler_params=_SC_PARAMS,
)

BF2 = 128
FW = 64
NCH2 = EPW // BF2


def _prop256_body(gcat_hbm, pk4_hbm, out_hbm,
                  pk_v, nrmbuf, idxcS, rows0, rows1, rowsS, zbuf, shared,
                  sem0, sem1):
    c = lax.axis_index("c")
    s = lax.axis_index("s")
    w = _worker(c, s)

    def one_pass(p, carry):
        pltpu.sync_copy(pk4_hbm.at[p, w], pk_v)
        _zero_shared(FW, s, zbuf, shared)
        plsc.subcore_barrier()
        _run_edges(FW, BF2, NCH2, pk_v, gcat_hbm, shared, nrmbuf, idxcS,
                   (rows0, rows1), rowsS, (sem0, sem1))
        plsc.subcore_barrier()
        pltpu.sync_copy(shared.at[pl.ds(s * RPS, RPS)],
                        out_hbm.at[c, p, pl.ds(s * RPS, RPS)])
        plsc.subcore_barrier()
        return carry

    lax.fori_loop(0, 4, one_pass, 0)


_prop256 = pl.kernel(
    _prop256_body,
    out_type=jax.ShapeDtypeStruct((NC, 4, N, FW), jnp.float32),
    mesh=_MESH,
    scratch_types=[
        pltpu.VMEM((NCH2, 3, BF2), jnp.int32),
        pltpu.VMEM((BF2,), jnp.float32),
        pltpu.VMEM((BF2,), jnp.int32),
        pltpu.VMEM((BF2, FW), jnp.float32),
        pltpu.VMEM((BF2, FW), jnp.float32),
        pltpu.VMEM((BF2, FW), jnp.float32),
        pltpu.VMEM((ZR, FW), jnp.float32),
        pltpu.VMEM_SHARED((N, FW), jnp.float32),
        pltpu.SemaphoreType.DMA, pltpu.SemaphoreType.DMA,
    ],
    compiler_params=_SC_PARAMS,
)


def _deg_body(col_hbm, ew_hbm, out_hbm, idx_v, ew_v, deg_v, tmp_v, acc_v, shared):
    c = lax.axis_index("c")
    s = lax.axis_index("s")
    w = _worker(c, s)

    def z(i, carry):
        deg_v[pl.ds(i * 16, 16)] = jnp.zeros((16,), jnp.float32)
        return carry

    lax.fori_loop(0, NP // 16, z, 0)

    def chunk(j, carry):
        base = w * EPW + j * B
        pltpu.sync_copy(col_hbm.at[pl.ds(base, B)], idx_v)
        pltpu.sync_copy(ew_hbm.at[pl.ds(base, B)], ew_v)
        for i in range(B // 16):
            sl = pl.ds(i * 16, 16)
            plsc.addupdate_scatter(deg_v, [idx_v[sl]], ew_v[sl])
        return carry

    lax.fori_loop(0, NCH, chunk, 0)

    pltpu.sync_copy(deg_v, shared.at[s])
    plsc.subcore_barrier()

    def z2(i, carry):
        acc_v[pl.ds(i * 16, 16)] = jnp.zeros((16,), jnp.float32)
        return carry

    lax.fori_loop(0, CW // 16, z2, 0)
    for t in range(NS):
        pltpu.sync_copy(shared.at[t, pl.ds(s * CW, CW)], tmp_v)
        for i in range(CW // 16):
            sl = pl.ds(i * 16, 16)
            acc_v[sl] = acc_v[sl] + tmp_v[sl]
    pltpu.sync_copy(acc_v, out_hbm.at[c, pl.ds(s * CW, CW)])


_sc_deg = pl.kernel(
    _deg_body,
    out_type=jax.ShapeDtypeStruct((NC, NP), jnp.float32),
    mesh=_MESH,
    scratch_types=[
        pltpu.VMEM((B,), jnp.int32),
        pltpu.VMEM((B,), jnp.float32),
        pltpu.VMEM((NP,), jnp.float32),
        pltpu.VMEM((CW,), jnp.float32),
        pltpu.VMEM((CW,), jnp.float32),
        pltpu.VMEM_SHARED((NS, NP), jnp.float32),
    ],
    compiler_params=_SC_PARAMS,
)


def _norm_body(row_hbm, col_hbm, ew_hbm, dinv_hbm, out_hbm,
               idxr_v, idxc_v, ew_v, nrm_v, dinv_v):
    c = lax.axis_index("c")
    s = lax.axis_index("s")
    w = _worker(c, s)
    pltpu.sync_copy(dinv_hbm, dinv_v)

    def chunk(j, carry):
        base = w * EPW + j * B
        pltpu.sync_copy(row_hbm.at[pl.ds(base, B)], idxr_v)
        pltpu.sync_copy(col_hbm.at[pl.ds(base, B)], idxc_v)
        pltpu.sync_copy(ew_hbm.at[pl.ds(base, B)], ew_v)
        for i in range(B // 16):
            sl = pl.ds(i * 16, 16)
            vr = plsc.load_gather(dinv_v, [idxr_v[sl]])
            vc = plsc.load_gather(dinv_v, [idxc_v[sl]])
            nrm_v[sl] = vr * ew_v[sl] * vc
        pltpu.sync_copy(nrm_v, out_hbm.at[pl.ds(base, B)])
        return carry

    lax.fori_loop(0, NCH, chunk, 0)


_sc_norm = pl.kernel(
    _norm_body,
    out_type=jax.ShapeDtypeStruct((EP,), jnp.float32),
    mesh=_MESH,
    scratch_types=[
        pltpu.VMEM((B,), jnp.int32),
        pltpu.VMEM((B,), jnp.int32),
        pltpu.VMEM((B,), jnp.float32),
        pltpu.VMEM((B,), jnp.float32),
        pltpu.VMEM((NP,), jnp.float32),
    ],
    compiler_params=_SC_PARAMS,
)


def _prep_body(deg2_ref, dinv_ref, sn_ref):
    d = deg2_ref[0] + deg2_ref[1] + 1.0
    dinv_ref[...] = lax.rsqrt(d)
    sn_ref[...] = 1.0 / d


_tc_prep = pl.pallas_call(
    _prep_body,
    out_shape=[
        jax.ShapeDtypeStruct((NP // 128, 128), jnp.float32),
        jax.ShapeDtypeStruct((NP // 128, 128), jnp.float32),
    ],
)

BR = 400


def _head_body(x_ref, W1_ref, b1_ref, Wc_ref, g0_ref, g1_ref, g2_ref, g3_ref):
    h1 = jnp.dot(x_ref[...], W1_ref[...], preferred_element_type=jnp.float32)
    h1 = jnp.maximum(h1 + b1_ref[...], 0.0)
    g = jnp.dot(h1, Wc_ref[...], preferred_element_type=jnp.float32)
    for k, r in enumerate((g0_ref, g1_ref, g2_ref, g3_ref)):
        r[...] = g[:, k * 64:(k + 1) * 64]


_tc_head = pl.pallas_call(
    _head_body,
    grid=(N // BR,),
    in_specs=[
        pl.BlockSpec((BR, 128), lambda i: (i, 0)),
        pl.BlockSpec((128, 256), lambda i: (0, 0)),
        pl.BlockSpec((1, 256), lambda i: (0, 0)),
        pl.BlockSpec((256, 256), lambda i: (0, 0)),
    ],
    out_specs=[pl.BlockSpec((BR, 64), lambda i: (i, 0)) for _ in range(4)],
    out_shape=[jax.ShapeDtypeStruct((N, 64), jnp.float32) for _ in range(4)],
)


def _mid_body(p0_ref, p1_ref, p2_ref, p3_ref, g0_ref, g1_ref, g2_ref, g3_ref,
              sn_ref, bc_ref, W2_ref, b2_ref, o_ref):
    sn = sn_ref[...]
    parts = [p_ref[0, 0] + p_ref[1, 0] + sn * g_ref[...]
             for p_ref, g_ref in zip((p0_ref, p1_ref, p2_ref, p3_ref),
                                     (g0_ref, g1_ref, g2_ref, g3_ref))]
    h2 = jnp.maximum(jnp.concatenate(parts, axis=1) + bc_ref[...], 0.0)
    o_ref[...] = jnp.dot(h2, W2_ref[...], preferred_element_type=jnp.float32) + b2_ref[...]


_tc_mid = pl.pallas_call(
    _mid_body,
    grid=(N // BR,),
    in_specs=[
        pl.BlockSpec((2, 1, BR, 64), lambda i, k=k: (0, k, i, 0))
        for k in range(4)
    ] + [
        pl.BlockSpec((BR, 64), lambda i: (i, 0)) for _ in range(4)
    ] + [
        pl.BlockSpec((BR, 1), lambda i: (i, 0)),
        pl.BlockSpec((1, 256), lambda i: (0, 0)),
        pl.BlockSpec((256, 16), lambda i: (0, 0)),
        pl.BlockSpec((1, 16), lambda i: (0, 0)),
    ],
    out_specs=pl.BlockSpec((BR, 16), lambda i: (i, 0)),
    out_shape=jax.ShapeDtypeStruct((N, 16), jnp.float32),
)


def _axpy_body(p_ref, sn_ref, h_ref, x0_ref, o_ref):
    o_ref[...] = (1.0 - ALPHA) * (p_ref[0] + p_ref[1] + sn_ref[...] * h_ref[...]) \
        + ALPHA * x0_ref[...]


_tc_axpy = pl.pallas_call(
    _axpy_body,
    out_shape=jax.ShapeDtypeStruct((N * 16 // 128, 128), jnp.float32),
)


def _final_body(p_ref, sn_ref, h_ref, x0_ref, o_ref):
    t = (1.0 - ALPHA) * (p_ref[0] + p_ref[1] + sn_ref[...] * h_ref[...]) \
        + ALPHA * x0_ref[...]
    m = jnp.max(t, axis=-1, keepdims=True)
    e = jnp.exp(t - m)
    o_ref[...] = t - m - jnp.log(jnp.sum(e, axis=-1, keepdims=True))


_tc_final = pl.pallas_call(
    _final_body,
    out_shape=jax.ShapeDtypeStruct((N, 16), jnp.float32),
)


def kernel(x, edge_index, edge_weight, W1, b1, Wc, bc, W2, b2):
    row = edge_index[0]
    col = edge_index[1]
    pad = EP - E
    rowp = jnp.concatenate([row, jnp.zeros((pad,), jnp.int32)])
    colp = jnp.concatenate([col, jnp.zeros((pad,), jnp.int32)])
    ewp = jnp.concatenate([edge_weight, jnp.zeros((pad,), jnp.float32)])

    deg2 = _sc_deg(colp, ewp)
    dinv80, sn80 = _tc_prep(deg2.reshape(NC, NP // 128, 128))
    dinv = dinv80.reshape(NP)
    snf = sn80.reshape(NP)[:N]
    sn_col = snf[:, None]
    sn16 = jnp.broadcast_to(sn_col, (N, 16))
    sn16r = sn16.reshape(N * 16 // 128, 128)

    normp = _sc_norm(rowp, colp, ewp, dinv)
    nbits = lax.bitcast_convert_type(normp, jnp.int32)

    pk16 = jnp.stack([rowp.reshape(NW, NCH, B), colp.reshape(NW, NCH, B),
                      nbits.reshape(NW, NCH, B)], axis=2)
    rowp4 = rowp[None, :] + (jnp.arange(4, dtype=jnp.int32) * N)[:, None]
    c4 = jnp.broadcast_to(colp.reshape(1, NW, NCH2, BF2), (4, NW, NCH2, BF2))
    n4 = jnp.broadcast_to(nbits.reshape(1, NW, NCH2, BF2), (4, NW, NCH2, BF2))
    pk256 = jnp.stack([rowp4.reshape(4, NW, NCH2, BF2), c4, n4], axis=3)

    gq = _tc_head(x, W1, b1.reshape(1, -1), Wc)
    gcat = jnp.concatenate(gq, axis=0)
    p256 = _prop256(gcat, pk256)
    x0 = _tc_mid(p256, p256, p256, p256, gq[0], gq[1], gq[2], gq[3],
                 sn_col, bc.reshape(1, -1), W2, b2.reshape(1, -1))

    h = x0
    x0r = x0.reshape(N * 16 // 128, 128)
    for _ in range(K - 1):
        p = _prop16(h, pk16)
        hr = _tc_axpy(p.reshape(2, N * 16 // 128, 128), sn16r,
                      h.reshape(N * 16 // 128, 128), x0r)
        h = hr.reshape(N, 16)
    p = _prop16(h, pk16)
    return _tc_final(p, sn16, h, x0)


# speedup vs baseline: 13.0266x; 1.0490x over previous
import functools

import jax
import jax.numpy as jnp
from jax import lax
from jax.experimental import pallas as pl
from jax.experimental.pallas import tpu as pltpu
from jax.experimental.pallas import tpu_sc as plsc

N = 10000
E = 320000
K = 10
ALPHA = 0.1

NC = 2
NS = 16
NW = NC * NS
B = 128
NCH = 80
EPW = NCH * B
EP = NW * EPW
RPS = N // NS
ZR = 125
NP = 10240
CW = NP // NS

_MESH = plsc.VectorSubcoreMesh(
    core_axis_name="c", subcore_axis_name="s", num_cores=NC, num_subcores=NS
)
_SC_PARAMS = pltpu.CompilerParams(use_tc_tiling_on_sc=False, needs_layout_passes=False)


def _worker(c, s):
    return s * NC + c


def _run_edges(F, BF, nch, pk_v, gsrc, shared, nrmbuf, idxcS, rows, rowsS,
               semG, semW):
    def phase(j, p, first):
        q = 1 - p
        pltpu.make_async_copy(
            gsrc.at[pk_v.at[lax.rem(j, nch), 0]], rows[p], semG[p]).wait()
        pltpu.async_copy(
            gsrc.at[pk_v.at[lax.rem(j + 1, nch), 0]], rows[q], semG[q])
        for i in range(BF // 16):
            sl = pl.ds(i * 16, 16)
            nrmbuf[sl] = plsc.bitcast(pk_v[j, 2, sl], jnp.float32)
            idxcS[p][sl] = pk_v[j, 1, sl]
        for i in range(BF):
            sp = plsc.load_gather(nrmbuf, [jnp.full((16,), i, jnp.int32)])
            for qq in range(F // 16):
                sl = pl.ds(qq * 16, 16)
                rowsS[p][i, sl] = rows[p][i, sl] * sp
        if not first:
            pltpu.make_async_copy(rowsS[q], shared.at[idxcS[q]], semW).wait()
        pltpu.async_copy(rowsS[p], shared.at[idxcS[p]], semW, add=True)

    pltpu.async_copy(gsrc.at[pk_v.at[0, 0]], rows[0], semG[0])
    phase(0, 0, True)
    phase(1, 1, False)

    def steady(jj, carry):
        j = 2 + 2 * jj
        phase(j, 0, False)
        phase(j + 1, 1, False)
        return carry

    lax.fori_loop(0, (nch - 2) // 2, steady, 0)
    pltpu.make_async_copy(rowsS[1], shared.at[idxcS[1]], semW).wait()
    pltpu.make_async_copy(gsrc.at[pk_v.at[0, 0]], rows[0], semG[0]).wait()


def _zero_shared(F, s, zbuf, shared):
    def zb(i, carry):
        for q in range(F // 16):
            zbuf[i, pl.ds(q * 16, 16)] = jnp.zeros((16,), jnp.float32)
        return carry

    lax.fori_loop(0, ZR, zb, 0)
    for r in range(RPS // ZR):
        pltpu.sync_copy(zbuf, shared.at[pl.ds(s * RPS + r * ZR, ZR)])


def _prop16_body(h_hbm, pk_hbm, out_hbm,
                 pk_v, nrmbuf, idxcS0, idxcS1, rows0, rows1, rowsS0, rowsS1,
                 zbuf, shared, shared_h, sem0, sem1, semW):
    c = lax.axis_index("c")
    s = lax.axis_index("s")
    w = _worker(c, s)
    pltpu.sync_copy(pk_hbm.at[w], pk_v)
    pltpu.sync_copy(h_hbm.at[pl.ds(s * RPS, RPS)],
                    shared_h.at[pl.ds(s * RPS, RPS)])
    _zero_shared(16, s, zbuf, shared)
    plsc.subcore_barrier()
    _run_edges(16, B, NCH, pk_v, shared_h, shared, nrmbuf, (idxcS0, idxcS1),
               (rows0, rows1), (rowsS0, rowsS1), (sem0, sem1), semW)
    plsc.subcore_barrier()
    pltpu.sync_copy(shared.at[pl.ds(s * RPS, RPS)],
                    out_hbm.at[c, pl.ds(s * RPS, RPS)])


_prop16 = pl.kernel(
    _prop16_body,
    out_type=jax.ShapeDtypeStruct((NC, N, 16), jnp.float32),
    mesh=_MESH,
    scratch_types=[
        pltpu.VMEM((NCH, 3, B), jnp.int32),
        pltpu.VMEM((B,), jnp.float32),
        pltpu.VMEM((B,), jnp.int32),
        pltpu.VMEM((B,), jnp.int32),
        pltpu.VMEM((B, 16), jnp.float32),
        pltpu.VMEM((B, 16), jnp.float32),
        pltpu.VMEM((B, 16), jnp.float32),
        pltpu.VMEM((B, 16), jnp.float32),
        pltpu.VMEM((ZR, 16), jnp.float32),
        pltpu.VMEM_SHARED((N, 16), jnp.float32),
        pltpu.VMEM_SHARED((N, 16), jnp.float32),
        pltpu.SemaphoreType.DMA, pltpu.SemaphoreType.DMA,
        pltpu.SemaphoreType.DMA,
    ],
    compiler_params=_SC_PARAMS,
)

BF2 = 128
FW = 64
NCH2 = EPW // BF2


def _prop256_body(gcat_hbm, pk4_hbm, out_hbm,
                  pk_v, nrmbuf, idxcS0, idxcS1, rows0, rows1, rowsS0, rowsS1,
                  zbuf, shared, sem0, sem1, semW):
    c = lax.axis_index("c")
    s = lax.axis_index("s")
    w = _worker(c, s)

    def one_pass(p, carry):
        pltpu.sync_copy(pk4_hbm.at[p, w], pk_v)
        _zero_shared(FW, s, zbuf, shared)
        plsc.subcore_barrier()
        _run_edges(FW, BF2, NCH2, pk_v, gcat_hbm, shared, nrmbuf,
                   (idxcS0, idxcS1), (rows0, rows1), (rowsS0, rowsS1),
                   (sem0, sem1), semW)
        plsc.subcore_barrier()
        pltpu.sync_copy(shared.at[pl.ds(s * RPS, RPS)],
                        out_hbm.at[c, p, pl.ds(s * RPS, RPS)])
        plsc.subcore_barrier()
        return carry

    lax.fori_loop(0, 4, one_pass, 0)


_prop256 = pl.kernel(
    _prop256_body,
    out_type=jax.ShapeDtypeStruct((NC, 4, N, FW), jnp.float32),
    mesh=_MESH,
    scratch_types=[
        pltpu.VMEM((NCH2, 3, BF2), jnp.int32),
        pltpu.VMEM((BF2,), jnp.float32),
        pltpu.VMEM((BF2,), jnp.int32),
        pltpu.VMEM((BF2,), jnp.int32),
        pltpu.VMEM((BF2, FW), jnp.float32),
        pltpu.VMEM((BF2, FW), jnp.float32),
        pltpu.VMEM((BF2, FW), jnp.float32),
        pltpu.VMEM((BF2, FW), jnp.float32),
        pltpu.VMEM((ZR, FW), jnp.float32),
        pltpu.VMEM_SHARED((N, FW), jnp.float32),
        pltpu.SemaphoreType.DMA, pltpu.SemaphoreType.DMA,
        pltpu.SemaphoreType.DMA,
    ],
    compiler_params=_SC_PARAMS,
)


def _deg_body(col_hbm, ew_hbm, out_hbm, idx_v, ew_v, deg_v, tmp_v, acc_v, shared):
    c = lax.axis_index("c")
    s = lax.axis_index("s")
    w = _worker(c, s)

    def z(i, carry):
        deg_v[pl.ds(i * 16, 16)] = jnp.zeros((16,), jnp.float32)
        return carry

    lax.fori_loop(0, NP // 16, z, 0)

    def chunk(j, carry):
        base = w * EPW + j * B
        pltpu.sync_copy(col_hbm.at[pl.ds(base, B)], idx_v)
        pltpu.sync_copy(ew_hbm.at[pl.ds(base, B)], ew_v)
        for i in range(B // 16):
            sl = pl.ds(i * 16, 16)
            plsc.addupdate_scatter(deg_v, [idx_v[sl]], ew_v[sl])
        return carry

    lax.fori_loop(0, NCH, chunk, 0)

    pltpu.sync_copy(deg_v, shared.at[s])
    plsc.subcore_barrier()

    def z2(i, carry):
        acc_v[pl.ds(i * 16, 16)] = jnp.zeros((16,), jnp.float32)
        return carry

    lax.fori_loop(0, CW // 16, z2, 0)
    for t in range(NS):
        pltpu.sync_copy(shared.at[t, pl.ds(s * CW, CW)], tmp_v)
        for i in range(CW // 16):
            sl = pl.ds(i * 16, 16)
            acc_v[sl] = acc_v[sl] + tmp_v[sl]
    pltpu.sync_copy(acc_v, out_hbm.at[c, pl.ds(s * CW, CW)])


_sc_deg = pl.kernel(
    _deg_body,
    out_type=jax.ShapeDtypeStruct((NC, NP), jnp.float32),
    mesh=_MESH,
    scratch_types=[
        pltpu.VMEM((B,), jnp.int32),
        pltpu.VMEM((B,), jnp.float32),
        pltpu.VMEM((NP,), jnp.float32),
        pltpu.VMEM((CW,), jnp.float32),
        pltpu.VMEM((CW,), jnp.float32),
        pltpu.VMEM_SHARED((NS, NP), jnp.float32),
    ],
    compiler_params=_SC_PARAMS,
)


def _norm_body(row_hbm, col_hbm, ew_hbm, dinv_hbm, out_hbm,
               idxr_v, idxc_v, ew_v, nrm_v, dinv_v):
    c = lax.axis_index("c")
    s = lax.axis_index("s")
    w = _worker(c, s)
    pltpu.sync_copy(dinv_hbm, dinv_v)

    def chunk(j, carry):
        base = w * EPW + j * B
        pltpu.sync_copy(row_hbm.at[pl.ds(base, B)], idxr_v)
        pltpu.sync_copy(col_hbm.at[pl.ds(base, B)], idxc_v)
        pltpu.sync_copy(ew_hbm.at[pl.ds(base, B)], ew_v)
        for i in range(B // 16):
            sl = pl.ds(i * 16, 16)
            vr = plsc.load_gather(dinv_v, [idxr_v[sl]])
            vc = plsc.load_gather(dinv_v, [idxc_v[sl]])
            nrm_v[sl] = vr * ew_v[sl] * vc
        pltpu.sync_copy(nrm_v, out_hbm.at[pl.ds(base, B)])
        return carry

    lax.fori_loop(0, NCH, chunk, 0)


_sc_norm = pl.kernel(
    _norm_body,
    out_type=jax.ShapeDtypeStruct((EP,), jnp.float32),
    mesh=_MESH,
    scratch_types=[
        pltpu.VMEM((B,), jnp.int32),
        pltpu.VMEM((B,), jnp.int32),
        pltpu.VMEM((B,), jnp.float32),
        pltpu.VMEM((B,), jnp.float32),
        pltpu.VMEM((NP,), jnp.float32),
    ],
    compiler_params=_SC_PARAMS,
)


def _prep_body(deg2_ref, dinv_ref, sn_ref):
    d = deg2_ref[0] + deg2_ref[1] + 1.0
    dinv_ref[...] = lax.rsqrt(d)
    sn_ref[...] = 1.0 / d


_tc_prep = pl.pallas_call(
    _prep_body,
    out_shape=[
        jax.ShapeDtypeStruct((NP // 128, 128), jnp.float32),
        jax.ShapeDtypeStruct((NP // 128, 128), jnp.float32),
    ],
)

BR = 400


def _head_body(x_ref, W1_ref, b1_ref, Wc_ref, g0_ref, g1_ref, g2_ref, g3_ref):
    h1 = jnp.dot(x_ref[...], W1_ref[...], preferred_element_type=jnp.float32)
    h1 = jnp.maximum(h1 + b1_ref[...], 0.0)
    g = jnp.dot(h1, Wc_ref[...], preferred_element_type=jnp.float32)
    for k, r in enumerate((g0_ref, g1_ref, g2_ref, g3_ref)):
        r[...] = g[:, k * 64:(k + 1) * 64]


_tc_head = pl.pallas_call(
    _head_body,
    grid=(N // BR,),
    in_specs=[
        pl.BlockSpec((BR, 128), lambda i: (i, 0)),
        pl.BlockSpec((128, 256), lambda i: (0, 0)),
        pl.BlockSpec((1, 256), lambda i: (0, 0)),
        pl.BlockSpec((256, 256), lambda i: (0, 0)),
    ],
    out_specs=[pl.BlockSpec((BR, 64), lambda i: (i, 0)) for _ in range(4)],
    out_shape=[jax.ShapeDtypeStruct((N, 64), jnp.float32) for _ in range(4)],
)


def _mid_body(p0_ref, p1_ref, p2_ref, p3_ref, g0_ref, g1_ref, g2_ref, g3_ref,
              sn_ref, bc_ref, W2_ref, b2_ref, o_ref):
    sn = sn_ref[...]
    parts = [p_ref[0, 0] + p_ref[1, 0] + sn * g_ref[...]
             for p_ref, g_ref in zip((p0_ref, p1_ref, p2_ref, p3_ref),
                                     (g0_ref, g1_ref, g2_ref, g3_ref))]
    h2 = jnp.maximum(jnp.concatenate(parts, axis=1) + bc_ref[...], 0.0)
    o_ref[...] = jnp.dot(h2, W2_ref[...], preferred_element_type=jnp.float32) + b2_ref[...]


_tc_mid = pl.pallas_call(
    _mid_body,
    grid=(N // BR,),
    in_specs=[
        pl.BlockSpec((2, 1, BR, 64), lambda i, k=k: (0, k, i, 0))
        for k in range(4)
    ] + [
        pl.BlockSpec((BR, 64), lambda i: (i, 0)) for _ in range(4)
    ] + [
        pl.BlockSpec((BR, 1), lambda i: (i, 0)),
        pl.BlockSpec((1, 256), lambda i: (0, 0)),
        pl.BlockSpec((256, 16), lambda i: (0, 0)),
        pl.BlockSpec((1, 16), lambda i: (0, 0)),
    ],
    out_specs=pl.BlockSpec((BR, 16), lambda i: (i, 0)),
    out_shape=jax.ShapeDtypeStruct((N, 16), jnp.float32),
)


def _axpy_body(p_ref, sn_ref, h_ref, x0_ref, o_ref):
    o_ref[...] = (1.0 - ALPHA) * (p_ref[0] + p_ref[1] + sn_ref[...] * h_ref[...]) \
        + ALPHA * x0_ref[...]


_tc_axpy = pl.pallas_call(
    _axpy_body,
    out_shape=jax.ShapeDtypeStruct((N * 16 // 128, 128), jnp.float32),
)


def _final_body(p_ref, sn_ref, h_ref, x0_ref, o_ref):
    t = (1.0 - ALPHA) * (p_ref[0] + p_ref[1] + sn_ref[...] * h_ref[...]) \
        + ALPHA * x0_ref[...]
    m = jnp.max(t, axis=-1, keepdims=True)
    e = jnp.exp(t - m)
    o_ref[...] = t - m - jnp.log(jnp.sum(e, axis=-1, keepdims=True))


_tc_final = pl.pallas_call(
    _final_body,
    out_shape=jax.ShapeDtypeStruct((N, 16), jnp.float32),
)


def kernel(x, edge_index, edge_weight, W1, b1, Wc, bc, W2, b2):
    row = edge_index[0]
    col = edge_index[1]
    pad = EP - E
    rowp = jnp.concatenate([row, jnp.zeros((pad,), jnp.int32)])
    colp = jnp.concatenate([col, jnp.zeros((pad,), jnp.int32)])
    ewp = jnp.concatenate([edge_weight, jnp.zeros((pad,), jnp.float32)])

    deg2 = _sc_deg(colp, ewp)
    dinv80, sn80 = _tc_prep(deg2.reshape(NC, NP // 128, 128))
    dinv = dinv80.reshape(NP)
    snf = sn80.reshape(NP)[:N]
    sn_col = snf[:, None]
    sn16 = jnp.broadcast_to(sn_col, (N, 16))
    sn16r = sn16.reshape(N * 16 // 128, 128)

    normp = _sc_norm(rowp, colp, ewp, dinv)
    nbits = lax.bitcast_convert_type(normp, jnp.int32)

    pk16 = jnp.stack([rowp.reshape(NW, NCH, B), colp.reshape(NW, NCH, B),
                      nbits.reshape(NW, NCH, B)], axis=2)
    rowp4 = rowp[None, :] + (jnp.arange(4, dtype=jnp.int32) * N)[:, None]
    c4 = jnp.broadcast_to(colp.reshape(1, NW, NCH2, BF2), (4, NW, NCH2, BF2))
    n4 = jnp.broadcast_to(nbits.reshape(1, NW, NCH2, BF2), (4, NW, NCH2, BF2))
    pk256 = jnp.stack([rowp4.reshape(4, NW, NCH2, BF2), c4, n4], axis=3)

    gq = _tc_head(x, W1, b1.reshape(1, -1), Wc)
    gcat = jnp.concatenate(gq, axis=0)
    p256 = _prop256(gcat, pk256)
    x0 = _tc_mid(p256, p256, p256, p256, gq[0], gq[1], gq[2], gq[3],
                 sn_col, bc.reshape(1, -1), W2, b2.reshape(1, -1))

    h = x0
    x0r = x0.reshape(N * 16 // 128, 128)
    for _ in range(K - 1):
        p = _prop16(h, pk16)
        hr = _tc_axpy(p.reshape(2, N * 16 // 128, 128), sn16r,
                      h.reshape(N * 16 // 128, 128), x0r)
        h = hr.reshape(N, 16)
    p = _prop16(h, pk16)
    return _tc_final(p, sn16, h, x0)


# speedup vs baseline: 14.2047x; 1.0904x over previous
import functools

import jax
import jax.numpy as jnp
from jax import lax
from jax.experimental import pallas as pl
from jax.experimental.pallas import tpu as pltpu
from jax.experimental.pallas import tpu_sc as plsc

N = 10000
E = 320000
K = 10
ALPHA = 0.1

NC = 2
NS = 16
NW = NC * NS
B = 128
NCH = 80
EPW = NCH * B
EP = NW * EPW
RPS = N // NS
ZR = 125
NP = 10240
CW = NP // NS

_MESH = plsc.VectorSubcoreMesh(
    core_axis_name="c", subcore_axis_name="s", num_cores=NC, num_subcores=NS
)
_SC_PARAMS = pltpu.CompilerParams(use_tc_tiling_on_sc=False, needs_layout_passes=False)


def _worker(c, s):
    return s * NC + c


def _run_edges(F, BF, nch, pk_v, gsrc, shared, nrmbuf, idxcS, rows, rowsS,
               semG, semW):
    def phase(j, p, first):
        q = 1 - p
        pltpu.make_async_copy(
            gsrc.at[pk_v.at[lax.rem(j, nch), 0]], rows[p], semG[p]).wait()
        pltpu.async_copy(
            gsrc.at[pk_v.at[lax.rem(j + 1, nch), 0]], rows[q], semG[q])
        for i in range(BF // 16):
            sl = pl.ds(i * 16, 16)
            nrmbuf[sl] = plsc.bitcast(pk_v[j, 2, sl], jnp.float32)
            idxcS[p][sl] = pk_v[j, 1, sl]
        for i in range(BF):
            sp = plsc.load_gather(nrmbuf, [jnp.full((16,), i, jnp.int32)])
            for qq in range(F // 16):
                sl = pl.ds(qq * 16, 16)
                rowsS[p][i, sl] = rows[p][i, sl] * sp
        if not first:
            pltpu.make_async_copy(rowsS[q], shared.at[idxcS[q]], semW).wait()
        pltpu.async_copy(rowsS[p], shared.at[idxcS[p]], semW, add=True)

    pltpu.async_copy(gsrc.at[pk_v.at[0, 0]], rows[0], semG[0])
    phase(0, 0, True)
    phase(1, 1, False)

    def steady(jj, carry):
        j = 2 + 2 * jj
        phase(j, 0, False)
        phase(j + 1, 1, False)
        return carry

    lax.fori_loop(0, (nch - 2) // 2, steady, 0)
    pltpu.make_async_copy(rowsS[1], shared.at[idxcS[1]], semW).wait()
    pltpu.make_async_copy(gsrc.at[pk_v.at[0, 0]], rows[0], semG[0]).wait()


def _zero_shared(F, s, zbuf, shared):
    def zb(i, carry):
        for q in range(F // 16):
            zbuf[i, pl.ds(q * 16, 16)] = jnp.zeros((16,), jnp.float32)
        return carry

    lax.fori_loop(0, ZR, zb, 0)
    for r in range(RPS // ZR):
        pltpu.sync_copy(zbuf, shared.at[pl.ds(s * RPS + r * ZR, ZR)])


def _prop16_body(h_hbm, pk_hbm, out_hbm,
                 pk_v, nrmbuf, idxcS0, idxcS1, rows0, rows1, rowsS0, rowsS1,
                 zbuf, shared, shared_h, sem0, sem1, semW):
    c = lax.axis_index("c")
    s = lax.axis_index("s")
    w = _worker(c, s)
    pltpu.sync_copy(pk_hbm.at[w], pk_v)
    pltpu.sync_copy(h_hbm.at[pl.ds(s * RPS, RPS)],
                    shared_h.at[pl.ds(s * RPS, RPS)])
    _zero_shared(16, s, zbuf, shared)
    plsc.subcore_barrier()
    _run_edges(16, B, NCH, pk_v, shared_h, shared, nrmbuf, (idxcS0, idxcS1),
               (rows0, rows1), (rowsS0, rowsS1), (sem0, sem1), semW)
    plsc.subcore_barrier()
    pltpu.sync_copy(shared.at[pl.ds(s * RPS, RPS)],
                    out_hbm.at[c, pl.ds(s * RPS, RPS)])


_prop16 = pl.kernel(
    _prop16_body,
    out_type=jax.ShapeDtypeStruct((NC, N, 16), jnp.float32),
    mesh=_MESH,
    scratch_types=[
        pltpu.VMEM((NCH, 3, B), jnp.int32),
        pltpu.VMEM((B,), jnp.float32),
        pltpu.VMEM((B,), jnp.int32),
        pltpu.VMEM((B,), jnp.int32),
        pltpu.VMEM((B, 16), jnp.float32),
        pltpu.VMEM((B, 16), jnp.float32),
        pltpu.VMEM((B, 16), jnp.float32),
        pltpu.VMEM((B, 16), jnp.float32),
        pltpu.VMEM((ZR, 16), jnp.float32),
        pltpu.VMEM_SHARED((N, 16), jnp.float32),
        pltpu.VMEM_SHARED((N, 16), jnp.float32),
        pltpu.SemaphoreType.DMA, pltpu.SemaphoreType.DMA,
        pltpu.SemaphoreType.DMA,
    ],
    compiler_params=_SC_PARAMS,
)

BF2 = 128
FW = 64
NCH2 = EPW // BF2


def _prop256_body(gcat_hbm, pk4_hbm, out_hbm,
                  pk_v, nrmbuf, idxcS0, idxcS1, rows0, rows1, rowsS0, rowsS1,
                  zbuf, shared, sem0, sem1, semW):
    c = lax.axis_index("c")
    s = lax.axis_index("s")
    w = _worker(c, s)

    def one_pass(p, carry):
        pltpu.sync_copy(pk4_hbm.at[p, w], pk_v)
        _zero_shared(FW, s, zbuf, shared)
        plsc.subcore_barrier()
        _run_edges(FW, BF2, NCH2, pk_v, gcat_hbm, shared, nrmbuf,
                   (idxcS0, idxcS1), (rows0, rows1), (rowsS0, rowsS1),
                   (sem0, sem1), semW)
        plsc.subcore_barrier()
        pltpu.sync_copy(shared.at[pl.ds(s * RPS, RPS)],
                        out_hbm.at[c, p, pl.ds(s * RPS, RPS)])
        plsc.subcore_barrier()
        return carry

    lax.fori_loop(0, 4, one_pass, 0)


_prop256 = pl.kernel(
    _prop256_body,
    out_type=jax.ShapeDtypeStruct((NC, 4, N, FW), jnp.float32),
    mesh=_MESH,
    scratch_types=[
        pltpu.VMEM((NCH2, 3, BF2), jnp.int32),
        pltpu.VMEM((BF2,), jnp.float32),
        pltpu.VMEM((BF2,), jnp.int32),
        pltpu.VMEM((BF2,), jnp.int32),
        pltpu.VMEM((BF2, FW), jnp.float32),
        pltpu.VMEM((BF2, FW), jnp.float32),
        pltpu.VMEM((BF2, FW), jnp.float32),
        pltpu.VMEM((BF2, FW), jnp.float32),
        pltpu.VMEM((ZR, FW), jnp.float32),
        pltpu.VMEM_SHARED((N, FW), jnp.float32),
        pltpu.SemaphoreType.DMA, pltpu.SemaphoreType.DMA,
        pltpu.SemaphoreType.DMA,
    ],
    compiler_params=_SC_PARAMS,
)


def _deg_body(pke_hbm, out_hbm, pk_v, deg_v, tmp_v, acc_v, shared):
    c = lax.axis_index("c")
    s = lax.axis_index("s")
    w = _worker(c, s)
    pltpu.sync_copy(pke_hbm.at[w], pk_v)

    def z(i, carry):
        deg_v[pl.ds(i * 16, 16)] = jnp.zeros((16,), jnp.float32)
        return carry

    lax.fori_loop(0, NP // 16, z, 0)

    def chunk(j, carry):
        for i in range(B // 16):
            sl = pl.ds(i * 16, 16)
            plsc.addupdate_scatter(
                deg_v, [pk_v[j, 1, sl]],
                plsc.bitcast(pk_v[j, 2, sl], jnp.float32))
        return carry

    lax.fori_loop(0, NCH, chunk, 0)

    pltpu.sync_copy(deg_v, shared.at[s])
    plsc.subcore_barrier()

    def z2(i, carry):
        acc_v[pl.ds(i * 16, 16)] = jnp.zeros((16,), jnp.float32)
        return carry

    lax.fori_loop(0, CW // 16, z2, 0)
    for t in range(NS):
        pltpu.sync_copy(shared.at[t, pl.ds(s * CW, CW)], tmp_v)
        for i in range(CW // 16):
            sl = pl.ds(i * 16, 16)
            acc_v[sl] = acc_v[sl] + tmp_v[sl]
    pltpu.sync_copy(acc_v, out_hbm.at[c, pl.ds(s * CW, CW)])


_sc_deg = pl.kernel(
    _deg_body,
    out_type=jax.ShapeDtypeStruct((NC, NP), jnp.float32),
    mesh=_MESH,
    scratch_types=[
        pltpu.VMEM((NCH, 3, B), jnp.int32),
        pltpu.VMEM((NP,), jnp.float32),
        pltpu.VMEM((CW,), jnp.float32),
        pltpu.VMEM((CW,), jnp.float32),
        pltpu.VMEM_SHARED((NS, NP), jnp.float32),
    ],
    compiler_params=_SC_PARAMS,
)


def _norm_body(pke_hbm, dinv_hbm, out_hbm, pk_v, nrm_v, dinv_v):
    c = lax.axis_index("c")
    s = lax.axis_index("s")
    w = _worker(c, s)
    pltpu.sync_copy(pke_hbm.at[w], pk_v)
    pltpu.sync_copy(dinv_hbm, dinv_v)

    def chunk(j, carry):
        base = w * EPW + j * B
        for i in range(B // 16):
            sl = pl.ds(i * 16, 16)
            vr = plsc.load_gather(dinv_v, [pk_v[j, 0, sl]])
            vc = plsc.load_gather(dinv_v, [pk_v[j, 1, sl]])
            nrm_v[sl] = vr * plsc.bitcast(pk_v[j, 2, sl], jnp.float32) * vc
        pltpu.sync_copy(nrm_v, out_hbm.at[pl.ds(base, B)])
        return carry

    lax.fori_loop(0, NCH, chunk, 0)


_sc_norm = pl.kernel(
    _norm_body,
    out_type=jax.ShapeDtypeStruct((EP,), jnp.float32),
    mesh=_MESH,
    scratch_types=[
        pltpu.VMEM((NCH, 3, B), jnp.int32),
        pltpu.VMEM((B,), jnp.float32),
        pltpu.VMEM((NP,), jnp.float32),
    ],
    compiler_params=_SC_PARAMS,
)


def _prep_body(deg2_ref, dinv_ref, sn_ref):
    d = deg2_ref[0] + deg2_ref[1] + 1.0
    dinv_ref[...] = lax.rsqrt(d)
    sn_ref[...] = 1.0 / d


_tc_prep = pl.pallas_call(
    _prep_body,
    out_shape=[
        jax.ShapeDtypeStruct((NP // 128, 128), jnp.float32),
        jax.ShapeDtypeStruct((NP // 128, 128), jnp.float32),
    ],
)

BR = 400


def _head_body(x_ref, W1_ref, b1_ref, Wc_ref, g0_ref, g1_ref, g2_ref, g3_ref):
    h1 = jnp.dot(x_ref[...], W1_ref[...], preferred_element_type=jnp.float32)
    h1 = jnp.maximum(h1 + b1_ref[...], 0.0)
    g = jnp.dot(h1, Wc_ref[...], preferred_element_type=jnp.float32)
    for k, r in enumerate((g0_ref, g1_ref, g2_ref, g3_ref)):
        r[...] = g[:, k * 64:(k + 1) * 64]


_tc_head = pl.pallas_call(
    _head_body,
    grid=(N // BR,),
    in_specs=[
        pl.BlockSpec((BR, 128), lambda i: (i, 0)),
        pl.BlockSpec((128, 256), lambda i: (0, 0)),
        pl.BlockSpec((1, 256), lambda i: (0, 0)),
        pl.BlockSpec((256, 256), lambda i: (0, 0)),
    ],
    out_specs=[pl.BlockSpec((BR, 64), lambda i: (i, 0)) for _ in range(4)],
    out_shape=[jax.ShapeDtypeStruct((N, 64), jnp.float32) for _ in range(4)],
)


def _mid_body(p0_ref, p1_ref, p2_ref, p3_ref, g0_ref, g1_ref, g2_ref, g3_ref,
              sn_ref, bc_ref, W2_ref, b2_ref, o_ref):
    sn = sn_ref[...]
    parts = [p_ref[0, 0] + p_ref[1, 0] + sn * g_ref[...]
             for p_ref, g_ref in zip((p0_ref, p1_ref, p2_ref, p3_ref),
                                     (g0_ref, g1_ref, g2_ref, g3_ref))]
    h2 = jnp.maximum(jnp.concatenate(parts, axis=1) + bc_ref[...], 0.0)
    o_ref[...] = jnp.dot(h2, W2_ref[...], preferred_element_type=jnp.float32) + b2_ref[...]


_tc_mid = pl.pallas_call(
    _mid_body,
    grid=(N // BR,),
    in_specs=[
        pl.BlockSpec((2, 1, BR, 64), lambda i, k=k: (0, k, i, 0))
        for k in range(4)
    ] + [
        pl.BlockSpec((BR, 64), lambda i: (i, 0)) for _ in range(4)
    ] + [
        pl.BlockSpec((BR, 1), lambda i: (i, 0)),
        pl.BlockSpec((1, 256), lambda i: (0, 0)),
        pl.BlockSpec((256, 16), lambda i: (0, 0)),
        pl.BlockSpec((1, 16), lambda i: (0, 0)),
    ],
    out_specs=pl.BlockSpec((BR, 16), lambda i: (i, 0)),
    out_shape=jax.ShapeDtypeStruct((N, 16), jnp.float32),
)


def _axpy_body(p_ref, sn_ref, h_ref, x0_ref, o_ref):
    o_ref[...] = (1.0 - ALPHA) * (p_ref[0] + p_ref[1] + sn_ref[...] * h_ref[...]) \
        + ALPHA * x0_ref[...]


_tc_axpy = pl.pallas_call(
    _axpy_body,
    out_shape=jax.ShapeDtypeStruct((N * 16 // 128, 128), jnp.float32),
)


def _final_body(p_ref, sn_ref, h_ref, x0_ref, o_ref):
    t = (1.0 - ALPHA) * (p_ref[0] + p_ref[1] + sn_ref[...] * h_ref[...]) \
        + ALPHA * x0_ref[...]
    m = jnp.max(t, axis=-1, keepdims=True)
    e = jnp.exp(t - m)
    o_ref[...] = t - m - jnp.log(jnp.sum(e, axis=-1, keepdims=True))


_tc_final = pl.pallas_call(
    _final_body,
    out_shape=jax.ShapeDtypeStruct((N, 16), jnp.float32),
)


def kernel(x, edge_index, edge_weight, W1, b1, Wc, bc, W2, b2):
    row = edge_index[0]
    col = edge_index[1]
    pad = EP - E
    rowp = jnp.concatenate([row, jnp.zeros((pad,), jnp.int32)])
    colp = jnp.concatenate([col, jnp.zeros((pad,), jnp.int32)])
    ewp = jnp.concatenate([edge_weight, jnp.zeros((pad,), jnp.float32)])

    ebits = lax.bitcast_convert_type(ewp, jnp.int32)
    pke = jnp.stack([rowp.reshape(NW, NCH, B), colp.reshape(NW, NCH, B),
                     ebits.reshape(NW, NCH, B)], axis=2)

    deg2 = _sc_deg(pke)
    dinv80, sn80 = _tc_prep(deg2.reshape(NC, NP // 128, 128))
    dinv = dinv80.reshape(NP)
    snf = sn80.reshape(NP)[:N]
    sn_col = snf[:, None]
    sn16 = jnp.broadcast_to(sn_col, (N, 16))
    sn16r = sn16.reshape(N * 16 // 128, 128)

    normp = _sc_norm(pke, dinv)
    nbits = lax.bitcast_convert_type(normp, jnp.int32)

    pk16 = jnp.stack([rowp.reshape(NW, NCH, B), colp.reshape(NW, NCH, B),
                      nbits.reshape(NW, NCH, B)], axis=2)
    rowp4 = rowp[None, :] + (jnp.arange(4, dtype=jnp.int32) * N)[:, None]
    c4 = jnp.broadcast_to(colp.reshape(1, NW, NCH2, BF2), (4, NW, NCH2, BF2))
    n4 = jnp.broadcast_to(nbits.reshape(1, NW, NCH2, BF2), (4, NW, NCH2, BF2))
    pk256 = jnp.stack([rowp4.reshape(4, NW, NCH2, BF2), c4, n4], axis=3)

    gq = _tc_head(x, W1, b1.reshape(1, -1), Wc)
    gcat = jnp.concatenate(gq, axis=0)
    p256 = _prop256(gcat, pk256)
    x0 = _tc_mid(p256, p256, p256, p256, gq[0], gq[1], gq[2], gq[3],
                 sn_col, bc.reshape(1, -1), W2, b2.reshape(1, -1))

    h = x0
    x0r = x0.reshape(N * 16 // 128, 128)
    for _ in range(K - 1):
        p = _prop16(h, pk16)
        hr = _tc_axpy(p.reshape(2, N * 16 // 128, 128), sn16r,
                      h.reshape(N * 16 // 128, 128), x0r)
        h = hr.reshape(N, 16)
    p = _prop16(h, pk16)
    return _tc_final(p, sn16, h, x0)


# speedup vs baseline: 14.5341x; 1.0232x over previous
import functools

import jax
import jax.numpy as jnp
from jax import lax
from jax.experimental import pallas as pl
from jax.experimental.pallas import tpu as pltpu
from jax.experimental.pallas import tpu_sc as plsc

N = 10000
E = 320000
K = 10
ALPHA = 0.1

NC = 2
NS = 16
NW = NC * NS
B = 128
NCH = 80
EPW = NCH * B
EP = NW * EPW
RPS = N // NS
ZR = 125
NP = 10240
CW = NP // NS

_MESH = plsc.VectorSubcoreMesh(
    core_axis_name="c", subcore_axis_name="s", num_cores=NC, num_subcores=NS
)
_SC_PARAMS = pltpu.CompilerParams(use_tc_tiling_on_sc=False, needs_layout_passes=False)


def _worker(c, s):
    return s * NC + c


def _run_edges(F, BF, nch, pk_v, gsrc, shared, nrmbuf, idxcS, rows, rowsS,
               semG, semW):
    def phase(j, p, first):
        q = (p + 1) % 3
        r2 = (p + 2) % 3
        pltpu.make_async_copy(
            gsrc.at[pk_v.at[lax.rem(j, nch), 0]], rows[p], semG[p]).wait()
        pltpu.async_copy(
            gsrc.at[pk_v.at[lax.rem(j + 2, nch), 0]], rows[r2], semG[r2])
        for i in range(BF // 16):
            sl = pl.ds(i * 16, 16)
            nrmbuf[sl] = plsc.bitcast(pk_v[j, 2, sl], jnp.float32)
            idxcS[p][sl] = pk_v[j, 1, sl]
        for i in range(BF):
            sp = plsc.load_gather(nrmbuf, [jnp.full((16,), i, jnp.int32)])
            for qq in range(F // 16):
                sl = pl.ds(qq * 16, 16)
                rowsS[p][i, sl] = rows[p][i, sl] * sp
        if not first:
            pltpu.make_async_copy(rowsS[r2], shared.at[idxcS[r2]], semW).wait()
        pltpu.async_copy(rowsS[p], shared.at[idxcS[p]], semW, add=True)

    pltpu.async_copy(gsrc.at[pk_v.at[0, 0]], rows[0], semG[0])
    pltpu.async_copy(gsrc.at[pk_v.at[1, 0]], rows[1], semG[1])
    phase(0, 0, True)
    phase(1, 1, False)

    def steady(jj, carry):
        j = 2 + 3 * jj
        phase(j, 2, False)
        phase(j + 1, 0, False)
        phase(j + 2, 1, False)
        return carry

    lax.fori_loop(0, (nch - 2) // 3, steady, 0)
    pltpu.make_async_copy(rowsS[(nch - 1) % 3],
                          shared.at[idxcS[(nch - 1) % 3]], semW).wait()
    pltpu.make_async_copy(gsrc.at[pk_v.at[0, 0]], rows[nch % 3],
                          semG[nch % 3]).wait()
    pltpu.make_async_copy(gsrc.at[pk_v.at[1, 0]], rows[(nch + 1) % 3],
                          semG[(nch + 1) % 3]).wait()


def _zero_shared(F, s, zbuf, shared):
    def zb(i, carry):
        for q in range(F // 16):
            zbuf[i, pl.ds(q * 16, 16)] = jnp.zeros((16,), jnp.float32)
        return carry

    lax.fori_loop(0, ZR, zb, 0)
    for r in range(RPS // ZR):
        pltpu.sync_copy(zbuf, shared.at[pl.ds(s * RPS + r * ZR, ZR)])


def _prop16_body(h_hbm, pk_hbm, out_hbm,
                 pk_v, nrmbuf, idxcS0, idxcS1, idxcS2, rows0, rows1, rows2,
                 rowsS0, rowsS1, rowsS2, zbuf, shared, shared_h,
                 sem0, sem1, sem2, semW):
    c = lax.axis_index("c")
    s = lax.axis_index("s")
    w = _worker(c, s)
    pltpu.sync_copy(pk_hbm.at[w], pk_v)
    pltpu.sync_copy(h_hbm.at[pl.ds(s * RPS, RPS)],
                    shared_h.at[pl.ds(s * RPS, RPS)])
    _zero_shared(16, s, zbuf, shared)
    plsc.subcore_barrier()
    _run_edges(16, B, NCH, pk_v, shared_h, shared, nrmbuf,
               (idxcS0, idxcS1, idxcS2), (rows0, rows1, rows2),
               (rowsS0, rowsS1, rowsS2), (sem0, sem1, sem2), semW)
    plsc.subcore_barrier()
    pltpu.sync_copy(shared.at[pl.ds(s * RPS, RPS)],
                    out_hbm.at[c, pl.ds(s * RPS, RPS)])


_prop16 = pl.kernel(
    _prop16_body,
    out_type=jax.ShapeDtypeStruct((NC, N, 16), jnp.float32),
    mesh=_MESH,
    scratch_types=[
        pltpu.VMEM((NCH, 3, B), jnp.int32),
        pltpu.VMEM((B,), jnp.float32),
        pltpu.VMEM((B,), jnp.int32),
        pltpu.VMEM((B,), jnp.int32),
        pltpu.VMEM((B,), jnp.int32),
        pltpu.VMEM((B, 16), jnp.float32),
        pltpu.VMEM((B, 16), jnp.float32),
        pltpu.VMEM((B, 16), jnp.float32),
        pltpu.VMEM((B, 16), jnp.float32),
        pltpu.VMEM((B, 16), jnp.float32),
        pltpu.VMEM((B, 16), jnp.float32),
        pltpu.VMEM((ZR, 16), jnp.float32),
        pltpu.VMEM_SHARED((N, 16), jnp.float32),
        pltpu.VMEM_SHARED((N, 16), jnp.float32),
        pltpu.SemaphoreType.DMA, pltpu.SemaphoreType.DMA,
        pltpu.SemaphoreType.DMA, pltpu.SemaphoreType.DMA,
    ],
    compiler_params=_SC_PARAMS,
)

BF2 = 128
FW = 64
NCH2 = EPW // BF2


def _prop256_body(gcat_hbm, pk4_hbm, out_hbm,
                  pk_v, nrmbuf, idxcS0, idxcS1, idxcS2, rows0, rows1, rows2,
                  rowsS0, rowsS1, rowsS2, zbuf, shared, sem0, sem1, sem2,
                  semW):
    c = lax.axis_index("c")
    s = lax.axis_index("s")
    w = _worker(c, s)

    def one_pass(p, carry):
        pltpu.sync_copy(pk4_hbm.at[p, w], pk_v)
        _zero_shared(FW, s, zbuf, shared)
        plsc.subcore_barrier()
        _run_edges(FW, BF2, NCH2, pk_v, gcat_hbm, shared, nrmbuf,
                   (idxcS0, idxcS1, idxcS2), (rows0, rows1, rows2),
                   (rowsS0, rowsS1, rowsS2), (sem0, sem1, sem2), semW)
        plsc.subcore_barrier()
        pltpu.sync_copy(shared.at[pl.ds(s * RPS, RPS)],
                        out_hbm.at[c, p, pl.ds(s * RPS, RPS)])
        plsc.subcore_barrier()
        return carry

    lax.fori_loop(0, 4, one_pass, 0)


_prop256 = pl.kernel(
    _prop256_body,
    out_type=jax.ShapeDtypeStruct((NC, 4, N, FW), jnp.float32),
    mesh=_MESH,
    scratch_types=[
        pltpu.VMEM((NCH2, 3, BF2), jnp.int32),
        pltpu.VMEM((BF2,), jnp.float32),
        pltpu.VMEM((BF2,), jnp.int32),
        pltpu.VMEM((BF2,), jnp.int32),
        pltpu.VMEM((BF2,), jnp.int32),
        pltpu.VMEM((BF2, FW), jnp.float32),
        pltpu.VMEM((BF2, FW), jnp.float32),
        pltpu.VMEM((BF2, FW), jnp.float32),
        pltpu.VMEM((BF2, FW), jnp.float32),
        pltpu.VMEM((BF2, FW), jnp.float32),
        pltpu.VMEM((BF2, FW), jnp.float32),
        pltpu.VMEM((ZR, FW), jnp.float32),
        pltpu.VMEM_SHARED((N, FW), jnp.float32),
        pltpu.SemaphoreType.DMA, pltpu.SemaphoreType.DMA,
        pltpu.SemaphoreType.DMA, pltpu.SemaphoreType.DMA,
    ],
    compiler_params=_SC_PARAMS,
)


def _deg_body(pke_hbm, out_hbm, pk_v, deg_v, tmp_v, acc_v, shared):
    c = lax.axis_index("c")
    s = lax.axis_index("s")
    w = _worker(c, s)
    pltpu.sync_copy(pke_hbm.at[w], pk_v)

    def z(i, carry):
        deg_v[pl.ds(i * 16, 16)] = jnp.zeros((16,), jnp.float32)
        return carry

    lax.fori_loop(0, NP // 16, z, 0)

    def chunk(j, carry):
        for i in range(B // 16):
            sl = pl.ds(i * 16, 16)
            plsc.addupdate_scatter(
                deg_v, [pk_v[j, 1, sl]],
                plsc.bitcast(pk_v[j, 2, sl], jnp.float32))
        return carry

    lax.fori_loop(0, NCH, chunk, 0)

    pltpu.sync_copy(deg_v, shared.at[s])
    plsc.subcore_barrier()

    def z2(i, carry):
        acc_v[pl.ds(i * 16, 16)] = jnp.zeros((16,), jnp.float32)
        return carry

    lax.fori_loop(0, CW // 16, z2, 0)
    for t in range(NS):
        pltpu.sync_copy(shared.at[t, pl.ds(s * CW, CW)], tmp_v)
        for i in range(CW // 16):
            sl = pl.ds(i * 16, 16)
            acc_v[sl] = acc_v[sl] + tmp_v[sl]
    pltpu.sync_copy(acc_v, out_hbm.at[c, pl.ds(s * CW, CW)])


_sc_deg = pl.kernel(
    _deg_body,
    out_type=jax.ShapeDtypeStruct((NC, NP), jnp.float32),
    mesh=_MESH,
    scratch_types=[
        pltpu.VMEM((NCH, 3, B), jnp.int32),
        pltpu.VMEM((NP,), jnp.float32),
        pltpu.VMEM((CW,), jnp.float32),
        pltpu.VMEM((CW,), jnp.float32),
        pltpu.VMEM_SHARED((NS, NP), jnp.float32),
    ],
    compiler_params=_SC_PARAMS,
)


def _norm_body(pke_hbm, dinv_hbm, out_hbm, pk_v, nrm_v, dinv_v):
    c = lax.axis_index("c")
    s = lax.axis_index("s")
    w = _worker(c, s)
    pltpu.sync_copy(pke_hbm.at[w], pk_v)
    pltpu.sync_copy(dinv_hbm, dinv_v)

    def chunk(j, carry):
        base = w * EPW + j * B
        for i in range(B // 16):
            sl = pl.ds(i * 16, 16)
            vr = plsc.load_gather(dinv_v, [pk_v[j, 0, sl]])
            vc = plsc.load_gather(dinv_v, [pk_v[j, 1, sl]])
            nrm_v[sl] = vr * plsc.bitcast(pk_v[j, 2, sl], jnp.float32) * vc
        pltpu.sync_copy(nrm_v, out_hbm.at[pl.ds(base, B)])
        return carry

    lax.fori_loop(0, NCH, chunk, 0)


_sc_norm = pl.kernel(
    _norm_body,
    out_type=jax.ShapeDtypeStruct((EP,), jnp.float32),
    mesh=_MESH,
    scratch_types=[
        pltpu.VMEM((NCH, 3, B), jnp.int32),
        pltpu.VMEM((B,), jnp.float32),
        pltpu.VMEM((NP,), jnp.float32),
    ],
    compiler_params=_SC_PARAMS,
)


def _prep_body(deg2_ref, dinv_ref, sn_ref):
    d = deg2_ref[0] + deg2_ref[1] + 1.0
    dinv_ref[...] = lax.rsqrt(d)
    sn_ref[...] = 1.0 / d


_tc_prep = pl.pallas_call(
    _prep_body,
    out_shape=[
        jax.ShapeDtypeStruct((NP // 128, 128), jnp.float32),
        jax.ShapeDtypeStruct((NP // 128, 128), jnp.float32),
    ],
)

BR = 400


def _head_body(x_ref, W1_ref, b1_ref, Wc_ref, g0_ref, g1_ref, g2_ref, g3_ref):
    h1 = jnp.dot(x_ref[...], W1_ref[...], preferred_element_type=jnp.float32)
    h1 = jnp.maximum(h1 + b1_ref[...], 0.0)
    g = jnp.dot(h1, Wc_ref[...], preferred_element_type=jnp.float32)
    for k, r in enumerate((g0_ref, g1_ref, g2_ref, g3_ref)):
        r[...] = g[:, k * 64:(k + 1) * 64]


_tc_head = pl.pallas_call(
    _head_body,
    grid=(N // BR,),
    in_specs=[
        pl.BlockSpec((BR, 128), lambda i: (i, 0)),
        pl.BlockSpec((128, 256), lambda i: (0, 0)),
        pl.BlockSpec((1, 256), lambda i: (0, 0)),
        pl.BlockSpec((256, 256), lambda i: (0, 0)),
    ],
    out_specs=[pl.BlockSpec((BR, 64), lambda i: (i, 0)) for _ in range(4)],
    out_shape=[jax.ShapeDtypeStruct((N, 64), jnp.float32) for _ in range(4)],
)


def _mid_body(p0_ref, p1_ref, p2_ref, p3_ref, g0_ref, g1_ref, g2_ref, g3_ref,
              sn_ref, bc_ref, W2_ref, b2_ref, o_ref):
    sn = sn_ref[...]
    parts = [p_ref[0, 0] + p_ref[1, 0] + sn * g_ref[...]
             for p_ref, g_ref in zip((p0_ref, p1_ref, p2_ref, p3_ref),
                                     (g0_ref, g1_ref, g2_ref, g3_ref))]
    h2 = jnp.maximum(jnp.concatenate(parts, axis=1) + bc_ref[...], 0.0)
    o_ref[...] = jnp.dot(h2, W2_ref[...], preferred_element_type=jnp.float32) + b2_ref[...]


_tc_mid = pl.pallas_call(
    _mid_body,
    grid=(N // BR,),
    in_specs=[
        pl.BlockSpec((2, 1, BR, 64), lambda i, k=k: (0, k, i, 0))
        for k in range(4)
    ] + [
        pl.BlockSpec((BR, 64), lambda i: (i, 0)) for _ in range(4)
    ] + [
        pl.BlockSpec((BR, 1), lambda i: (i, 0)),
        pl.BlockSpec((1, 256), lambda i: (0, 0)),
        pl.BlockSpec((256, 16), lambda i: (0, 0)),
        pl.BlockSpec((1, 16), lambda i: (0, 0)),
    ],
    out_specs=pl.BlockSpec((BR, 16), lambda i: (i, 0)),
    out_shape=jax.ShapeDtypeStruct((N, 16), jnp.float32),
)


def _axpy_body(p_ref, sn_ref, h_ref, x0_ref, o_ref):
    o_ref[...] = (1.0 - ALPHA) * (p_ref[0] + p_ref[1] + sn_ref[...] * h_ref[...]) \
        + ALPHA * x0_ref[...]


_tc_axpy = pl.pallas_call(
    _axpy_body,
    out_shape=jax.ShapeDtypeStruct((N * 16 // 128, 128), jnp.float32),
)


def _final_body(p_ref, sn_ref, h_ref, x0_ref, o_ref):
    t = (1.0 - ALPHA) * (p_ref[0] + p_ref[1] + sn_ref[...] * h_ref[...]) \
        + ALPHA * x0_ref[...]
    m = jnp.max(t, axis=-1, keepdims=True)
    e = jnp.exp(t - m)
    o_ref[...] = t - m - jnp.log(jnp.sum(e, axis=-1, keepdims=True))


_tc_final = pl.pallas_call(
    _final_body,
    out_shape=jax.ShapeDtypeStruct((N, 16), jnp.float32),
)


def kernel(x, edge_index, edge_weight, W1, b1, Wc, bc, W2, b2):
    row = edge_index[0]
    col = edge_index[1]
    pad = EP - E
    rowp = jnp.concatenate([row, jnp.zeros((pad,), jnp.int32)])
    colp = jnp.concatenate([col, jnp.zeros((pad,), jnp.int32)])
    ewp = jnp.concatenate([edge_weight, jnp.zeros((pad,), jnp.float32)])

    ebits = lax.bitcast_convert_type(ewp, jnp.int32)
    pke = jnp.stack([rowp.reshape(NW, NCH, B), colp.reshape(NW, NCH, B),
                     ebits.reshape(NW, NCH, B)], axis=2)

    deg2 = _sc_deg(pke)
    dinv80, sn80 = _tc_prep(deg2.reshape(NC, NP // 128, 128))
    dinv = dinv80.reshape(NP)
    snf = sn80.reshape(NP)[:N]
    sn_col = snf[:, None]
    sn16 = jnp.broadcast_to(sn_col, (N, 16))
    sn16r = sn16.reshape(N * 16 // 128, 128)

    normp = _sc_norm(pke, dinv)
    nbits = lax.bitcast_convert_type(normp, jnp.int32)

    pk16 = jnp.stack([rowp.reshape(NW, NCH, B), colp.reshape(NW, NCH, B),
                      nbits.reshape(NW, NCH, B)], axis=2)
    rowp4 = rowp[None, :] + (jnp.arange(4, dtype=jnp.int32) * N)[:, None]
    c4 = jnp.broadcast_to(colp.reshape(1, NW, NCH2, BF2), (4, NW, NCH2, BF2))
    n4 = jnp.broadcast_to(nbits.reshape(1, NW, NCH2, BF2), (4, NW, NCH2, BF2))
    pk256 = jnp.stack([rowp4.reshape(4, NW, NCH2, BF2), c4, n4], axis=3)

    gq = _tc_head(x, W1, b1.reshape(1, -1), Wc)
    gcat = jnp.concatenate(gq, axis=0)
    p256 = _prop256(gcat, pk256)
    x0 = _tc_mid(p256, p256, p256, p256, gq[0], gq[1], gq[2], gq[3],
                 sn_col, bc.reshape(1, -1), W2, b2.reshape(1, -1))

    h = x0
    x0r = x0.reshape(N * 16 // 128, 128)
    for _ in range(K - 1):
        p = _prop16(h, pk16)
        hr = _tc_axpy(p.reshape(2, N * 16 // 128, 128), sn16r,
                      h.reshape(N * 16 // 128, 128), x0r)
        h = hr.reshape(N, 16)
    p = _prop16(h, pk16)
    return _tc_final(p, sn16, h, x0)
